# Initial kernel scaffold; baseline (speedup 1.0000x reference)
import jax
import jax.numpy as jnp
from jax.experimental import pallas as pl


def kernel(op_feats, device_feats, tensor_feats, link_feats, placement_feats, params, op_types, prev_edges, succ_edges, link_edges, place_edges, serve_edges):
    raise NotImplementedError("write your pallas kernel here")


# speedup vs baseline: 1.1757x; 1.1757x over previous
import functools

import jax
import jax.numpy as jnp
from jax import lax
from jax.experimental import pallas as pl
from jax.experimental.pallas import tpu as pltpu
from jax.experimental.pallas import tpu_sc as plsc

N_OP = 50000
N_DEV = 256
E_T = 800000
E_L = 4096
E_P = 200000
D_OP = 56
D_DEV = 16
D_E = 16
H = 64
EH = 8
VOCAB = 128
EMB = 8

BO = 1024
NOP_P = 50176
GO = NOP_P // BO
ETP = 802816
EPP = 204800
ELP = 4096
BF = 4096

K = 128
HALF = NOP_P // 2
ACC_OP_ROWS = HALF + 16
DUMMY_OP = HALF
ACC_DEV_ROWS = 272
DUMMY_DEV = 256
PAD_DST = 1 << 30

F32 = jnp.float32
HI = lax.Precision.HIGHEST


def _full(shape):
    return pl.BlockSpec(shape, lambda *_: tuple(0 for _ in shape))


def _rows(shape):
    rank = len(shape)
    return pl.BlockSpec(shape, lambda i: (i,) + (0,) * (rank - 1))


def _oph_body(xf_ref, t_ref, emb_ref, w1_ref, w2_ref, b_ref, o_ref):
    t = t_ref[0, 0, :]
    iot = lax.broadcasted_iota(jnp.int32, (BO, VOCAB), 1)
    oh = (t[:, None] == iot).astype(F32)
    ew = jnp.dot(emb_ref[...], w2_ref[...], preferred_element_type=F32, precision=HI)
    acc = jnp.dot(xf_ref[...], w1_ref[...], preferred_element_type=F32, precision=HI)
    acc = acc + jnp.dot(oh, ew, preferred_element_type=F32, precision=HI) + b_ref[...]
    o_ref[...] = jnp.tanh(acc)


def _devh_body(x_ref, w_ref, b_ref, o_ref):
    o_ref[...] = jnp.tanh(
        jnp.dot(x_ref[...], w_ref[...], preferred_element_type=F32, precision=HI) + b_ref[...])


def _f2_body(x_ref, wa_ref, ba_ref, wb_ref, bb_ref, wfa_ref, bfa_ref,
             wfb_ref, bfb_ref, oa0, oa1, oa2, ob0, ob1, ob2):
    x = x_ref[...]
    efa = jnp.tanh(jnp.dot(x, wa_ref[...], preferred_element_type=F32, precision=HI) + ba_ref[...])
    efb = jnp.tanh(jnp.dot(x, wb_ref[...], preferred_element_type=F32, precision=HI) + bb_ref[...])
    outs_a = (oa0, oa1, oa2)
    outs_b = (ob0, ob1, ob2)
    for l in range(3):
        outs_a[l][...] = jnp.dot(efa, wfa_ref[l], preferred_element_type=F32, precision=HI) + bfa_ref[l]
        outs_b[l][...] = jnp.dot(efb, wfb_ref[l], preferred_element_type=F32, precision=HI) + bfb_ref[l]


def _f1_body(x_ref, wa_ref, ba_ref, wfa_ref, bfa_ref, oa0, oa1, oa2):
    x = x_ref[...]
    efa = jnp.tanh(jnp.dot(x, wa_ref[...], preferred_element_type=F32, precision=HI) + ba_ref[...])
    outs_a = (oa0, oa1, oa2)
    for l in range(3):
        outs_a[l][...] = jnp.dot(efa, wfa_ref[l], preferred_element_type=F32, precision=HI) + bfa_ref[l]


def _aop_body(h_ref, w1_ref, w2_ref, w3_ref, o1, o2, o3):
    h = h_ref[...]
    o1[...] = jnp.dot(h, w1_ref[...], preferred_element_type=F32, precision=HI)
    o2[...] = jnp.dot(h, w2_ref[...], preferred_element_type=F32, precision=HI)
    o3[...] = jnp.dot(h, w3_ref[...], preferred_element_type=F32, precision=HI)


def _adev_body(h_ref, w1_ref, w2_ref, o1, o2):
    h = h_ref[...]
    o1[...] = jnp.dot(h, w1_ref[...], preferred_element_type=F32, precision=HI)
    o2[...] = jnp.dot(h, w2_ref[...], preferred_element_type=F32, precision=HI)


def _updop_body(h_ref, su_ref, w_ref, b_ref, o_ref):
    o_ref[...] = h_ref[...] + (
        jnp.dot(su_ref[...], w_ref[...], preferred_element_type=F32, precision=HI) + b_ref[...])


def _upddev_body(h_ref, su_ref, w_ref, b_ref, o_ref):
    su = su_ref[0] + su_ref[1]
    o_ref[...] = h_ref[...] + (
        jnp.dot(su, w_ref[...], preferred_element_type=F32, precision=HI) + b_ref[...])


def _final_body(h_ref, d_ref, o_ref):
    o_ref[...] = lax.dot_general(
        h_ref[...], d_ref[...], (((1,), (1,)), ((), ())),
        preferred_element_type=F32, precision=HI)


def _sc_body(a_prev, a_succ, a_place, a_serve, a_link,
             f_prev, f_succ, f_place, f_serve, f_link,
             sp, dp, su, du, spl, dpl, sv, dv, sl, dl,
             op_out, dev_out,
             acc_op, acc_dev, src_buf, dst_buf, msg_buf, gat_buf, out_buf):
    c = lax.axis_index("c")
    s = lax.axis_index("s")
    base = c * HALF

    def _zrow(r, carry):
        z = jnp.zeros((16,), F32)
        for g in range(4):
            msg_buf[r, pl.ds(g * 16, 16)] = z
        return carry
    lax.fori_loop(0, K, _zrow, 0)
    row0 = s * (ACC_OP_ROWS // 16)
    for kk in range(12):
        pltpu.sync_copy(msg_buf, acc_op.at[pl.ds(row0 + kk * K, K)])
    pltpu.sync_copy(msg_buf.at[pl.ds(0, 33)], acc_op.at[pl.ds(row0 + 12 * K, 33)])
    pltpu.sync_copy(msg_buf.at[pl.ds(0, 17)], acc_dev.at[pl.ds(s * 17, 17)])
    plsc.subcore_barrier()

    def _process(a_ref, f_ref, src_ref, dst_ref, n_edges, op_dst):
        if op_dst:
            per = n_edges // 16
            start = s * per
        else:
            per = n_edges // 32
            start = (c * 16 + s) * per
        nch = per // K
        acc = acc_op if op_dst else acc_dev

        def _chunk(j, carry):
            off = start + j * K
            pltpu.sync_copy(src_ref.at[pl.ds(off, K)], src_buf)
            pltpu.sync_copy(dst_ref.at[pl.ds(off, K)], dst_buf)
            for g in range(8):
                d = dst_buf[pl.ds(g * 16, 16)]
                if op_dst:
                    dloc = d - base
                    m = (dloc >= 0) & (dloc < HALF)
                    d2 = jnp.where(m, dloc, DUMMY_OP)
                else:
                    d2 = jnp.where(d < N_DEV, d, DUMMY_DEV)
                dst_buf[pl.ds(g * 16, 16)] = d2
            pltpu.sync_copy(f_ref.at[pl.ds(off, K)], msg_buf)
            pltpu.sync_copy(a_ref.at[src_buf], gat_buf)

            def _row(r, rcarry):
                for g2 in range(4):
                    cols = pl.ds(g2 * 16, 16)
                    x = msg_buf[r, cols] + gat_buf[r, cols]
                    e = jnp.exp(x + x)
                    msg_buf[r, cols] = 1.0 - 2.0 / (e + 1.0)
                return rcarry
            lax.fori_loop(0, K, _row, 0)
            pltpu.sync_copy(msg_buf, acc.at[dst_buf], add=True)
            return carry
        lax.fori_loop(0, nch, _chunk, 0)

    _process(a_prev, f_prev, sp, dp, ETP, True)
    _process(a_succ, f_succ, su, du, ETP, True)
    _process(a_serve, f_serve, sv, dv, EPP, True)
    _process(a_place, f_place, spl, dpl, EPP, False)
    _process(a_link, f_link, sl, dl, ELP, False)

    plsc.subcore_barrier()

    r0 = s * (HALF // 16)

    def _wb(kk, carry):
        pltpu.sync_copy(acc_op.at[pl.ds(r0 + kk * 98, 98)], out_buf)
        pltpu.sync_copy(out_buf, op_out.at[pl.ds(base + r0 + kk * 98, 98)])
        return carry
    lax.fori_loop(0, 16, _wb, 0)
    pltpu.sync_copy(acc_dev.at[pl.ds(s * 16, 16)], out_buf.at[pl.ds(0, 16)])
    pltpu.sync_copy(out_buf.at[pl.ds(0, 16)], dev_out.at[c, pl.ds(s * 16, 16)])


def _make_sc_layer():
    mesh = plsc.VectorSubcoreMesh(
        core_axis_name="c", subcore_axis_name="s", num_cores=2, num_subcores=16)
    return pl.kernel(
        _sc_body,
        out_type=[
            jax.ShapeDtypeStruct((NOP_P, H), F32),
            jax.ShapeDtypeStruct((2, N_DEV, H), F32),
        ],
        mesh=mesh,
        compiler_params=pltpu.CompilerParams(use_tc_tiling_on_sc=False),
        scratch_types=[
            pltpu.VMEM_SHARED((ACC_OP_ROWS, H), F32),
            pltpu.VMEM_SHARED((ACC_DEV_ROWS, H), F32),
            pltpu.VMEM((K,), jnp.int32),
            pltpu.VMEM((K,), jnp.int32),
            pltpu.VMEM((K, H), F32),
            pltpu.VMEM((K, H), F32),
            pltpu.VMEM((98, H), F32),
        ],
    )


def _pad_rows(x, n):
    return jnp.pad(x, ((0, n - x.shape[0]),) + ((0, 0),) * (x.ndim - 1))


def _pad_edges(e, n):
    src = jnp.pad(e[0], (0, n - e.shape[1]))
    dst = jnp.pad(e[1], (0, n - e.shape[1]), constant_values=PAD_DST)
    return src.astype(jnp.int32), dst.astype(jnp.int32)


def kernel(op_feats, device_feats, tensor_feats, link_feats, placement_feats,
           params, op_types, prev_edges, succ_edges, link_edges, place_edges,
           serve_edges):
    p = params
    gconv = p["gconv"]

    opf = _pad_rows(op_feats, NOP_P)
    types3 = jnp.pad(op_types.astype(jnp.int32), (0, NOP_P - N_OP)).reshape(GO, 1, BO)
    tf = _pad_rows(tensor_feats, ETP)
    pf = _pad_rows(placement_feats, EPP)
    sp, dp = _pad_edges(prev_edges, ETP)
    su, du = _pad_edges(succ_edges, ETP)
    spl, dpl = _pad_edges(place_edges, EPP)
    sv, dv = _pad_edges(serve_edges, EPP)
    sl, dl = _pad_edges(link_edges, ELP)

    r1 = lambda b: b.reshape(1, -1)

    op_h = pl.pallas_call(
        _oph_body,
        grid=(GO,),
        in_specs=[_rows((BO, D_OP)), _rows((1, 1, BO)), _full((VOCAB, EMB)),
                  _full((D_OP, H)), _full((EMB, H)), _full((1, H))],
        out_specs=_rows((BO, H)),
        out_shape=jax.ShapeDtypeStruct((NOP_P, H), F32),
    )(opf, types3, p["emb"], p["op_trans"]["W"][:D_OP],
      p["op_trans"]["W"][D_OP:], r1(p["op_trans"]["b"]))

    dev_h = pl.pallas_call(
        _devh_body,
        in_specs=[_full((N_DEV, D_DEV)), _full((D_DEV, H)), _full((1, H))],
        out_specs=_full((N_DEV, H)),
        out_shape=jax.ShapeDtypeStruct((N_DEV, H), F32),
    )(device_feats, p["dev_trans"]["W"], r1(p["dev_trans"]["b"]))

    def stackW(et):
        return (jnp.stack([gconv[l][et]["W"][H:] for l in range(3)]),
                jnp.stack([gconv[l][et]["b"].reshape(1, H) for l in range(3)]))

    wf_prev, bf_prev = stackW("prev")
    wf_succ, bf_succ = stackW("succ")
    wf_place, bf_place = stackW("place")
    wf_serve, bf_serve = stackW("serve")
    wf_link, bf_link = stackW("link")

    et = p["edge_trans"]
    f_t = pl.pallas_call(
        _f2_body,
        grid=(ETP // BF,),
        in_specs=[_rows((BF, D_E)),
                  _full((D_E, EH)), _full((1, EH)), _full((D_E, EH)), _full((1, EH)),
                  _full((3, EH, H)), _full((3, 1, H)), _full((3, EH, H)), _full((3, 1, H))],
        out_specs=[_rows((BF, H))] * 6,
        out_shape=[jax.ShapeDtypeStruct((ETP, H), F32)] * 6,
    )(tf, et["prev"]["W"], r1(et["prev"]["b"]), et["succ"]["W"], r1(et["succ"]["b"]),
      wf_prev, bf_prev, wf_succ, bf_succ)
    f_prev, f_succ = f_t[:3], f_t[3:]

    f_p = pl.pallas_call(
        _f2_body,
        grid=(EPP // BF,),
        in_specs=[_rows((BF, D_E)),
                  _full((D_E, EH)), _full((1, EH)), _full((D_E, EH)), _full((1, EH)),
                  _full((3, EH, H)), _full((3, 1, H)), _full((3, EH, H)), _full((3, 1, H))],
        out_specs=[_rows((BF, H))] * 6,
        out_shape=[jax.ShapeDtypeStruct((EPP, H), F32)] * 6,
    )(pf, et["place"]["W"], r1(et["place"]["b"]), et["serve"]["W"], r1(et["serve"]["b"]),
      wf_place, bf_place, wf_serve, bf_serve)
    f_place, f_serve = f_p[:3], f_p[3:]

    f_link = pl.pallas_call(
        _f1_body,
        in_specs=[_full((ELP, D_E)), _full((D_E, EH)), _full((1, EH)),
                  _full((3, EH, H)), _full((3, 1, H))],
        out_specs=[_full((ELP, H))] * 3,
        out_shape=[jax.ShapeDtypeStruct((ELP, H), F32)] * 3,
    )(link_feats, et["link"]["W"], r1(et["link"]["b"]), wf_link, bf_link)

    sc_layer = _make_sc_layer()

    for l in range(3):
        lp = gconv[l]
        a_prev, a_succ, a_place = pl.pallas_call(
            _aop_body,
            grid=(GO,),
            in_specs=[_rows((BO, H))] + [_full((H, H))] * 3,
            out_specs=[_rows((BO, H))] * 3,
            out_shape=[jax.ShapeDtypeStruct((NOP_P, H), F32)] * 3,
        )(op_h, lp["prev"]["W"][:H], lp["succ"]["W"][:H], lp["place"]["W"][:H])

        a_serve, a_link = pl.pallas_call(
            _adev_body,
            in_specs=[_full((N_DEV, H))] + [_full((H, H))] * 2,
            out_specs=[_full((N_DEV, H))] * 2,
            out_shape=[jax.ShapeDtypeStruct((N_DEV, H), F32)] * 2,
        )(dev_h, lp["serve"]["W"][:H], lp["link"]["W"][:H])

        op_sum, dev_sum = sc_layer(
            a_prev, a_succ, a_place, a_serve, a_link,
            f_prev[l], f_succ[l], f_place[l], f_serve[l], f_link[l],
            sp, dp, su, du, spl, dpl, sv, dv, sl, dl)

        op_h = pl.pallas_call(
            _updop_body,
            grid=(GO,),
            in_specs=[_rows((BO, H)), _rows((BO, H)), _full((H, H)), _full((1, H))],
            out_specs=_rows((BO, H)),
            out_shape=jax.ShapeDtypeStruct((NOP_P, H), F32),
        )(op_h, op_sum, lp["op_final"]["W"], r1(lp["op_final"]["b"]))

        dev_h = pl.pallas_call(
            _upddev_body,
            in_specs=[_full((N_DEV, H)), _full((2, N_DEV, H)), _full((H, H)), _full((1, H))],
            out_specs=_full((N_DEV, H)),
            out_shape=jax.ShapeDtypeStruct((N_DEV, H), F32),
        )(dev_h, dev_sum, lp["device_final"]["W"], r1(lp["device_final"]["b"]))

    out = pl.pallas_call(
        _final_body,
        grid=(GO,),
        in_specs=[_rows((BO, H)), _full((N_DEV, H))],
        out_specs=_rows((BO, N_DEV)),
        out_shape=jax.ShapeDtypeStruct((NOP_P, N_DEV), F32),
    )(op_h, dev_h)
    return out[:N_OP]


# speedup vs baseline: 2.6881x; 2.2865x over previous
import functools

import jax
import jax.numpy as jnp
from jax import lax
from jax.experimental import pallas as pl
from jax.experimental.pallas import tpu as pltpu
from jax.experimental.pallas import tpu_sc as plsc

N_OP = 50000
N_DEV = 256
E_T = 800000
E_L = 4096
E_P = 200000
D_OP = 56
D_DEV = 16
D_E = 16
H = 64
EH = 8
VOCAB = 128
EMB = 8

BO = 1024
NOP_P = 50176
GO = NOP_P // BO
ETP = 802816
EPP = 204800
ELP = 4096
BF = 4096

KS = 128
KF = 128
HALF = NOP_P // 2
ACC_ROWS = 25376
DEVB = HALF + 16
DUMMY_OP = HALF
DUMMY_DEV = DEVB + 256
PAD_DST = 1 << 30

F32 = jnp.float32
HI = lax.Precision.HIGHEST


def _full(shape):
    return pl.BlockSpec(shape, lambda *_: tuple(0 for _ in shape))


def _rows(shape):
    rank = len(shape)
    return pl.BlockSpec(shape, lambda i: (i,) + (0,) * (rank - 1))


def _oph_body(xf_ref, t_ref, emb_ref, w1_ref, w2_ref, b_ref, o_ref):
    t = t_ref[0, 0, :]
    iot = lax.broadcasted_iota(jnp.int32, (BO, VOCAB), 1)
    oh = (t[:, None] == iot).astype(F32)
    ew = jnp.dot(emb_ref[...], w2_ref[...], preferred_element_type=F32, precision=HI)
    acc = jnp.dot(xf_ref[...], w1_ref[...], preferred_element_type=F32, precision=HI)
    acc = acc + jnp.dot(oh, ew, preferred_element_type=F32, precision=HI) + b_ref[...]
    o_ref[...] = jnp.tanh(acc)


def _devh_body(x_ref, w_ref, b_ref, o_ref):
    o_ref[...] = jnp.tanh(
        jnp.dot(x_ref[...], w_ref[...], preferred_element_type=F32, precision=HI) + b_ref[...])


def _f2_body(x_ref, wa_ref, ba_ref, wb_ref, bb_ref, wfa_ref, bfa_ref,
             wfb_ref, bfb_ref, oa0, oa1, oa2, ob0, ob1, ob2):
    x = x_ref[...]
    efa = jnp.tanh(jnp.dot(x, wa_ref[...], preferred_element_type=F32, precision=HI) + ba_ref[...])
    efb = jnp.tanh(jnp.dot(x, wb_ref[...], preferred_element_type=F32, precision=HI) + bb_ref[...])
    outs_a = (oa0, oa1, oa2)
    outs_b = (ob0, ob1, ob2)
    for l in range(3):
        outs_a[l][...] = jnp.dot(efa, wfa_ref[l], preferred_element_type=F32, precision=HI) + bfa_ref[l]
        outs_b[l][...] = jnp.dot(efb, wfb_ref[l], preferred_element_type=F32, precision=HI) + bfb_ref[l]


def _f1_body(x_ref, wa_ref, ba_ref, wfa_ref, bfa_ref, oa0, oa1, oa2):
    x = x_ref[...]
    efa = jnp.tanh(jnp.dot(x, wa_ref[...], preferred_element_type=F32, precision=HI) + ba_ref[...])
    outs_a = (oa0, oa1, oa2)
    for l in range(3):
        outs_a[l][...] = jnp.dot(efa, wfa_ref[l], preferred_element_type=F32, precision=HI) + bfa_ref[l]


def _aop_body(h_ref, w1_ref, w2_ref, w3_ref, o1, o2, o3):
    h = h_ref[...]
    o1[...] = jnp.dot(h, w1_ref[...], preferred_element_type=F32, precision=HI)
    o2[...] = jnp.dot(h, w2_ref[...], preferred_element_type=F32, precision=HI)
    o3[...] = jnp.dot(h, w3_ref[...], preferred_element_type=F32, precision=HI)


def _adev_body(h_ref, w1_ref, w2_ref, o1, o2):
    h = h_ref[...]
    o1[...] = jnp.dot(h, w1_ref[...], preferred_element_type=F32, precision=HI)
    o2[...] = jnp.dot(h, w2_ref[...], preferred_element_type=F32, precision=HI)


def _updop_body(h_ref, su_ref, w_ref, b_ref, o_ref):
    o_ref[...] = h_ref[...] + (
        jnp.dot(su_ref[...], w_ref[...], preferred_element_type=F32, precision=HI) + b_ref[...])


def _upddev_body(h_ref, su_ref, w_ref, b_ref, o_ref):
    su = su_ref[0] + su_ref[1]
    o_ref[...] = h_ref[...] + (
        jnp.dot(su, w_ref[...], preferred_element_type=F32, precision=HI) + b_ref[...])


def _final_body(h_ref, d_ref, o_ref):
    o_ref[...] = lax.dot_general(
        h_ref[...], d_ref[...], (((1,), (1,)), ((), ())),
        preferred_element_type=F32, precision=HI)


def _sc_body(a_prev, a_succ, a_place, a_serve, a_link,
             f_prev, f_succ, f_place, f_serve, f_link,
             sp, dp, su, du, spl, dpl, sv, dv, sl, dl,
             op_out, dev_out,
             acc, sbuf, dbuf, csrc, cdst, ceid,
             csrc_f, cdst_f, ceid_f, msg_buf, gat_buf, out_buf,
             sem_s0, sem_s1, sem_d0, sem_d1, sem_f, sem_a):
    c = lax.axis_index("c")
    s = lax.axis_index("s")
    base = c * HALF

    def _zrow(r, carry):
        z = jnp.zeros((16,), F32)
        for g in range(4):
            msg_buf[r, pl.ds(g * 16, 16)] = z
        return carry
    lax.fori_loop(0, KF, _zrow, 0)
    row0 = s * (ACC_ROWS // 16)
    for kk in range(12):
        pltpu.sync_copy(msg_buf, acc.at[pl.ds(row0 + kk * 128, 128)])
    pltpu.sync_copy(msg_buf.at[pl.ds(0, 50)], acc.at[pl.ds(row0 + 1536, 50)])
    plsc.subcore_barrier()

    lane = lax.broadcasted_iota(jnp.int32, (16,), 0)

    def _fire_start(a_ref, f_ref):
        for g in range(8):
            sl16 = pl.ds(g * 16, 16)
            csrc_f[sl16] = csrc[sl16]
            cdst_f[sl16] = cdst[sl16]
            ceid_f[sl16] = ceid[sl16]
        pltpu.async_copy(f_ref.at[ceid_f], msg_buf, sem_f)
        pltpu.async_copy(a_ref.at[csrc_f], gat_buf, sem_a)
        for g in range(8):
            lo = pl.ds(g * 16, 16)
            hi = pl.ds(KF + g * 16, 16)
            csrc[lo] = csrc[hi]
            cdst[lo] = cdst[hi]
            ceid[lo] = ceid[hi]

    def _fire_finish(a_ref, f_ref):
        pltpu.make_async_copy(f_ref.at[ceid_f], msg_buf, sem_f).wait()
        pltpu.make_async_copy(a_ref.at[csrc_f], gat_buf, sem_a).wait()

        def _row(r, rc):
            for g2 in range(4):
                cols = pl.ds(g2 * 16, 16)
                x = msg_buf[r, cols] + gat_buf[r, cols]
                e = jnp.exp(x + x)
                msg_buf[r, cols] = 1.0 - 2.0 / (e + 1.0)
            return rc
        lax.fori_loop(0, KF, _row, 0)
        pltpu.sync_copy(msg_buf, acc.at[cdst_f], add=True)

    def _process(a_ref, f_ref, src_ref, dst_ref, n_edges, op_dst):
        if op_dst:
            per = n_edges // 16
            start = s * per
        else:
            per = n_edges // 32
            start = (c * 16 + s) * per
        nch = per // KS

        def _scan(b, off, carry):
            fill, pending = carry
            for g in range(8):
                sl16 = pl.ds(g * 16, 16)
                d = dbuf[b, sl16]
                srcv = sbuf[b, sl16]
                if op_dst:
                    dloc = d - base
                    m = (dloc >= 0) & (dloc < HALF)
                else:
                    dloc = d + DEVB
                    m = (d >= 0) & (d < N_DEV)
                eidv = off + g * 16 + lane
                plsc.store_compressed(csrc.at[pl.ds(fill, 16)], srcv, mask=m)
                plsc.store_compressed(cdst.at[pl.ds(fill, 16)], dloc, mask=m)
                plsc.store_compressed(ceid.at[pl.ds(fill, 16)], eidv, mask=m)
                fill = fill + jnp.sum(m.astype(jnp.int32))
            do_fire = fill >= KF

            @pl.when(do_fire)
            def _():
                pl.when(pending == 1)(lambda: _fire_finish(a_ref, f_ref))
                _fire_start(a_ref, f_ref)
            return (jnp.where(do_fire, fill - KF, fill),
                    jnp.where(do_fire, 1, pending))

        def _start(b, off):
            sb, db = (sem_s0, sem_d0) if b == 0 else (sem_s1, sem_d1)
            pltpu.async_copy(src_ref.at[pl.ds(off, KS)], sbuf.at[b], sb)
            pltpu.async_copy(dst_ref.at[pl.ds(off, KS)], dbuf.at[b], db)

        def _wait(b):
            sb, db = (sem_s0, sem_d0) if b == 0 else (sem_s1, sem_d1)
            pltpu.make_async_copy(src_ref.at[pl.ds(0, KS)], sbuf.at[b], sb).wait()
            pltpu.make_async_copy(dst_ref.at[pl.ds(0, KS)], dbuf.at[b], db).wait()

        _start(0, start)
        zero2 = (jnp.int32(0), jnp.int32(0))
        if nch == 1:
            _wait(0)
            fill, pending = _scan(0, start, zero2)
        else:
            assert nch % 2 == 0

            def _pair(k, carry):
                off0 = start + (2 * k) * KS
                _wait(0)
                _start(1, off0 + KS)
                carry = _scan(0, off0, carry)
                _wait(1)
                pl.when(2 * k + 2 < nch)(lambda: _start(0, off0 + 2 * KS))
                carry = _scan(1, off0 + KS, carry)
                return carry
            fill, pending = lax.fori_loop(0, nch // 2, _pair, zero2)

        pl.when(pending == 1)(lambda: _fire_finish(a_ref, f_ref))
        fv = lax.broadcast(fill, (16,))
        dummy_d = DUMMY_OP if op_dst else DUMMY_DEV
        for g in range(8):
            sl16 = pl.ds(g * 16, 16)
            m2 = (g * 16 + lane) < fv
            csrc[sl16] = jnp.where(m2, csrc[sl16], 0)
            cdst[sl16] = jnp.where(m2, cdst[sl16], dummy_d)
            ceid[sl16] = jnp.where(m2, ceid[sl16], 0)
        _fire_start(a_ref, f_ref)
        _fire_finish(a_ref, f_ref)

    _process(a_prev, f_prev, sp, dp, ETP, True)
    _process(a_succ, f_succ, su, du, ETP, True)
    _process(a_serve, f_serve, sv, dv, EPP, True)
    _process(a_place, f_place, spl, dpl, EPP, False)
    _process(a_link, f_link, sl, dl, ELP, False)

    plsc.subcore_barrier()

    r0 = s * (HALF // 16)

    def _wb(kk, carry):
        pltpu.sync_copy(acc.at[pl.ds(r0 + kk * 98, 98)], out_buf)
        pltpu.sync_copy(out_buf, op_out.at[pl.ds(base + r0 + kk * 98, 98)])
        return carry
    lax.fori_loop(0, 16, _wb, 0)
    pltpu.sync_copy(acc.at[pl.ds(DEVB + s * 16, 16)], out_buf.at[pl.ds(0, 16)])
    pltpu.sync_copy(out_buf.at[pl.ds(0, 16)], dev_out.at[c, pl.ds(s * 16, 16)])


def _make_sc_layer():
    mesh = plsc.VectorSubcoreMesh(
        core_axis_name="c", subcore_axis_name="s", num_cores=2, num_subcores=16)
    return pl.kernel(
        _sc_body,
        out_type=[
            jax.ShapeDtypeStruct((NOP_P, H), F32),
            jax.ShapeDtypeStruct((2, N_DEV, H), F32),
        ],
        mesh=mesh,
        compiler_params=pltpu.CompilerParams(
            use_tc_tiling_on_sc=False, needs_layout_passes=False),
        scratch_types=[
            pltpu.VMEM_SHARED((ACC_ROWS, H), F32),
            pltpu.VMEM((2, KS), jnp.int32),
            pltpu.VMEM((2, KS), jnp.int32),
            pltpu.VMEM((2 * KF,), jnp.int32),
            pltpu.VMEM((2 * KF,), jnp.int32),
            pltpu.VMEM((2 * KF,), jnp.int32),
            pltpu.VMEM((KF,), jnp.int32),
            pltpu.VMEM((KF,), jnp.int32),
            pltpu.VMEM((KF,), jnp.int32),
            pltpu.VMEM((KF, H), F32),
            pltpu.VMEM((KF, H), F32),
            pltpu.VMEM((98, H), F32),
            pltpu.SemaphoreType.DMA,
            pltpu.SemaphoreType.DMA,
            pltpu.SemaphoreType.DMA,
            pltpu.SemaphoreType.DMA,
            pltpu.SemaphoreType.DMA,
            pltpu.SemaphoreType.DMA,
        ],
    )


def _pad_rows(x, n):
    return jnp.pad(x, ((0, n - x.shape[0]),) + ((0, 0),) * (x.ndim - 1))


def _pad_edges(e, n):
    src = jnp.pad(e[0], (0, n - e.shape[1]))
    dst = jnp.pad(e[1], (0, n - e.shape[1]), constant_values=PAD_DST)
    return src.astype(jnp.int32), dst.astype(jnp.int32)


def kernel(op_feats, device_feats, tensor_feats, link_feats, placement_feats,
           params, op_types, prev_edges, succ_edges, link_edges, place_edges,
           serve_edges):
    p = params
    gconv = p["gconv"]

    opf = _pad_rows(op_feats, NOP_P)
    types3 = jnp.pad(op_types.astype(jnp.int32), (0, NOP_P - N_OP)).reshape(GO, 1, BO)
    tf = _pad_rows(tensor_feats, ETP)
    pf = _pad_rows(placement_feats, EPP)
    sp, dp = _pad_edges(prev_edges, ETP)
    su, du = _pad_edges(succ_edges, ETP)
    spl, dpl = _pad_edges(place_edges, EPP)
    sv, dv = _pad_edges(serve_edges, EPP)
    sl, dl = _pad_edges(link_edges, ELP)

    r1 = lambda b: b.reshape(1, -1)

    op_h = pl.pallas_call(
        _oph_body,
        grid=(GO,),
        in_specs=[_rows((BO, D_OP)), _rows((1, 1, BO)), _full((VOCAB, EMB)),
                  _full((D_OP, H)), _full((EMB, H)), _full((1, H))],
        out_specs=_rows((BO, H)),
        out_shape=jax.ShapeDtypeStruct((NOP_P, H), F32),
    )(opf, types3, p["emb"], p["op_trans"]["W"][:D_OP],
      p["op_trans"]["W"][D_OP:], r1(p["op_trans"]["b"]))

    dev_h = pl.pallas_call(
        _devh_body,
        in_specs=[_full((N_DEV, D_DEV)), _full((D_DEV, H)), _full((1, H))],
        out_specs=_full((N_DEV, H)),
        out_shape=jax.ShapeDtypeStruct((N_DEV, H), F32),
    )(device_feats, p["dev_trans"]["W"], r1(p["dev_trans"]["b"]))

    def stackW(et):
        return (jnp.stack([gconv[l][et]["W"][H:] for l in range(3)]),
                jnp.stack([gconv[l][et]["b"].reshape(1, H) for l in range(3)]))

    wf_prev, bf_prev = stackW("prev")
    wf_succ, bf_succ = stackW("succ")
    wf_place, bf_place = stackW("place")
    wf_serve, bf_serve = stackW("serve")
    wf_link, bf_link = stackW("link")

    et = p["edge_trans"]
    f_t = pl.pallas_call(
        _f2_body,
        grid=(ETP // BF,),
        in_specs=[_rows((BF, D_E)),
                  _full((D_E, EH)), _full((1, EH)), _full((D_E, EH)), _full((1, EH)),
                  _full((3, EH, H)), _full((3, 1, H)), _full((3, EH, H)), _full((3, 1, H))],
        out_specs=[_rows((BF, H))] * 6,
        out_shape=[jax.ShapeDtypeStruct((ETP, H), F32)] * 6,
    )(tf, et["prev"]["W"], r1(et["prev"]["b"]), et["succ"]["W"], r1(et["succ"]["b"]),
      wf_prev, bf_prev, wf_succ, bf_succ)
    f_prev, f_succ = f_t[:3], f_t[3:]

    f_p = pl.pallas_call(
        _f2_body,
        grid=(EPP // BF,),
        in_specs=[_rows((BF, D_E)),
                  _full((D_E, EH)), _full((1, EH)), _full((D_E, EH)), _full((1, EH)),
                  _full((3, EH, H)), _full((3, 1, H)), _full((3, EH, H)), _full((3, 1, H))],
        out_specs=[_rows((BF, H))] * 6,
        out_shape=[jax.ShapeDtypeStruct((EPP, H), F32)] * 6,
    )(pf, et["place"]["W"], r1(et["place"]["b"]), et["serve"]["W"], r1(et["serve"]["b"]),
      wf_place, bf_place, wf_serve, bf_serve)
    f_place, f_serve = f_p[:3], f_p[3:]

    f_link = pl.pallas_call(
        _f1_body,
        in_specs=[_full((ELP, D_E)), _full((D_E, EH)), _full((1, EH)),
                  _full((3, EH, H)), _full((3, 1, H))],
        out_specs=[_full((ELP, H))] * 3,
        out_shape=[jax.ShapeDtypeStruct((ELP, H), F32)] * 3,
    )(link_feats, et["link"]["W"], r1(et["link"]["b"]), wf_link, bf_link)

    sc_layer = _make_sc_layer()

    for l in range(3):
        lp = gconv[l]
        a_prev, a_succ, a_place = pl.pallas_call(
            _aop_body,
            grid=(GO,),
            in_specs=[_rows((BO, H))] + [_full((H, H))] * 3,
            out_specs=[_rows((BO, H))] * 3,
            out_shape=[jax.ShapeDtypeStruct((NOP_P, H), F32)] * 3,
        )(op_h, lp["prev"]["W"][:H], lp["succ"]["W"][:H], lp["place"]["W"][:H])

        a_serve, a_link = pl.pallas_call(
            _adev_body,
            in_specs=[_full((N_DEV, H))] + [_full((H, H))] * 2,
            out_specs=[_full((N_DEV, H))] * 2,
            out_shape=[jax.ShapeDtypeStruct((N_DEV, H), F32)] * 2,
        )(dev_h, lp["serve"]["W"][:H], lp["link"]["W"][:H])

        op_sum, dev_sum = sc_layer(
            a_prev, a_succ, a_place, a_serve, a_link,
            f_prev[l], f_succ[l], f_place[l], f_serve[l], f_link[l],
            sp, dp, su, du, spl, dpl, sv, dv, sl, dl)

        op_h = pl.pallas_call(
            _updop_body,
            grid=(GO,),
            in_specs=[_rows((BO, H)), _rows((BO, H)), _full((H, H)), _full((1, H))],
            out_specs=_rows((BO, H)),
            out_shape=jax.ShapeDtypeStruct((NOP_P, H), F32),
        )(op_h, op_sum, lp["op_final"]["W"], r1(lp["op_final"]["b"]))

        dev_h = pl.pallas_call(
            _upddev_body,
            in_specs=[_full((N_DEV, H)), _full((2, N_DEV, H)), _full((H, H)), _full((1, H))],
            out_specs=_full((N_DEV, H)),
            out_shape=jax.ShapeDtypeStruct((N_DEV, H), F32),
        )(dev_h, dev_sum, lp["device_final"]["W"], r1(lp["device_final"]["b"]))

    out = pl.pallas_call(
        _final_body,
        grid=(GO,),
        in_specs=[_rows((BO, H)), _full((N_DEV, H))],
        out_specs=_rows((BO, N_DEV)),
        out_shape=jax.ShapeDtypeStruct((NOP_P, N_DEV), F32),
    )(op_h, dev_h)
    return out[:N_OP]


# speedup vs baseline: 3.7051x; 1.3783x over previous
import functools

import jax
import jax.numpy as jnp
from jax import lax
from jax.experimental import pallas as pl
from jax.experimental.pallas import tpu as pltpu
from jax.experimental.pallas import tpu_sc as plsc

N_OP = 50000
N_DEV = 256
E_T = 800000
E_L = 4096
E_P = 200000
D_OP = 56
D_DEV = 16
D_E = 16
H = 64
EH = 8
VOCAB = 128
EMB = 8

BO = 1024
NOP_P = 50176
GO = NOP_P // BO
ETP = 802816
EPP = 204800
ELP = 4096
BF = 4096

KS = 128
KF = 128
HALF = NOP_P // 2
ACC_ROWS = 25376
DEVB = HALF + 16
DUMMY_OP = HALF
DUMMY_DEV = DEVB + 256
PAD_DST = 1 << 30

F32 = jnp.float32
HI = lax.Precision.HIGHEST


def _full(shape):
    return pl.BlockSpec(shape, lambda *_: tuple(0 for _ in shape))


def _rows(shape):
    rank = len(shape)
    return pl.BlockSpec(shape, lambda i: (i,) + (0,) * (rank - 1))


def _oph_body(xf_ref, t_ref, emb_ref, w1_ref, w2_ref, b_ref, o_ref):
    t = t_ref[0, 0, :]
    iot = lax.broadcasted_iota(jnp.int32, (BO, VOCAB), 1)
    oh = (t[:, None] == iot).astype(F32)
    ew = jnp.dot(emb_ref[...], w2_ref[...], preferred_element_type=F32, precision=HI)
    acc = jnp.dot(xf_ref[...], w1_ref[...], preferred_element_type=F32, precision=HI)
    acc = acc + jnp.dot(oh, ew, preferred_element_type=F32, precision=HI) + b_ref[...]
    o_ref[...] = jnp.tanh(acc)


def _devh_body(x_ref, w_ref, b_ref, o_ref):
    o_ref[...] = jnp.tanh(
        jnp.dot(x_ref[...], w_ref[...], preferred_element_type=F32, precision=HI) + b_ref[...])


def _f2_body(x_ref, wa_ref, ba_ref, wb_ref, bb_ref, wfa_ref, bfa_ref,
             wfb_ref, bfb_ref, oa0, oa1, oa2, ob0, ob1, ob2):
    x = x_ref[...]
    efa = jnp.tanh(jnp.dot(x, wa_ref[...], preferred_element_type=F32, precision=HI) + ba_ref[...])
    efb = jnp.tanh(jnp.dot(x, wb_ref[...], preferred_element_type=F32, precision=HI) + bb_ref[...])
    outs_a = (oa0, oa1, oa2)
    outs_b = (ob0, ob1, ob2)
    for l in range(3):
        outs_a[l][...] = jnp.dot(efa, wfa_ref[l], preferred_element_type=F32, precision=HI) + bfa_ref[l]
        outs_b[l][...] = jnp.dot(efb, wfb_ref[l], preferred_element_type=F32, precision=HI) + bfb_ref[l]


def _f1_body(x_ref, wa_ref, ba_ref, wfa_ref, bfa_ref, oa0, oa1, oa2):
    x = x_ref[...]
    efa = jnp.tanh(jnp.dot(x, wa_ref[...], preferred_element_type=F32, precision=HI) + ba_ref[...])
    outs_a = (oa0, oa1, oa2)
    for l in range(3):
        outs_a[l][...] = jnp.dot(efa, wfa_ref[l], preferred_element_type=F32, precision=HI) + bfa_ref[l]


def _bd(w):
    a, b = w.shape
    z = jnp.zeros((a, b), w.dtype)
    return jnp.concatenate([jnp.concatenate([w, z], axis=1),
                            jnp.concatenate([z, w], axis=1)], axis=0)


def _b2(b):
    return jnp.concatenate([b, b], axis=-1)


def _aop_body(h_ref, w1_ref, w2_ref, w3_ref, o1, o2, o3):
    h = h_ref[...]
    o1[...] = jnp.dot(h, w1_ref[...], preferred_element_type=F32, precision=HI)
    o2[...] = jnp.dot(h, w2_ref[...], preferred_element_type=F32, precision=HI)
    o3[...] = jnp.dot(h, w3_ref[...], preferred_element_type=F32, precision=HI)


def _adev_body(h_ref, w1_ref, w2_ref, o1, o2):
    h = h_ref[...]
    o1[...] = jnp.dot(h, w1_ref[...], preferred_element_type=F32, precision=HI)
    o2[...] = jnp.dot(h, w2_ref[...], preferred_element_type=F32, precision=HI)


def _updop_body(h_ref, su_ref, w_ref, b_ref, o_ref):
    o_ref[...] = h_ref[...] + (
        jnp.dot(su_ref[...], w_ref[...], preferred_element_type=F32, precision=HI) + b_ref[...])


def _upddev_body(h_ref, su_ref, w_ref, b_ref, o_ref):
    su = su_ref[0] + su_ref[1]
    o_ref[...] = h_ref[...] + (
        jnp.dot(su, w_ref[...], preferred_element_type=F32, precision=HI) + b_ref[...])


def _final_body(h_ref, d_ref, o_ref):
    o_ref[...] = lax.dot_general(
        h_ref[...], d_ref[...], (((1,), (1,)), ((), ())),
        preferred_element_type=F32, precision=HI)


def _sc_body(a_prev, a_succ, a_place, a_serve, a_link,
             f_prev, f_succ, f_place, f_serve, f_link,
             sp, dp, su, du, spl, dpl, sv, dv, sl, dl,
             op_out, dev_out,
             acc, sbuf, dbuf, csrc, cdst, ceid,
             csrc_f, cdst_f, ceid_f, msg_buf, gat_buf, out_buf,
             sem_s0, sem_s1, sem_d0, sem_d1, sem_f, sem_a, sem_s2, sem_s3,
             sem_d2, sem_d3):
    c = lax.axis_index("c")
    s = lax.axis_index("s")
    base = c * HALF

    def _zrow(r, carry):
        z = jnp.zeros((16,), F32)
        for g in range(4):
            msg_buf[r, pl.ds(g * 16, 16)] = z
        return carry
    lax.fori_loop(0, KF, _zrow, 0)
    row0 = s * (ACC_ROWS // 16)
    for kk in range(12):
        pltpu.sync_copy(msg_buf, acc.at[pl.ds(row0 + kk * 128, 128)])
    pltpu.sync_copy(msg_buf.at[pl.ds(0, 50)], acc.at[pl.ds(row0 + 1536, 50)])
    plsc.subcore_barrier()

    lane = lax.broadcasted_iota(jnp.int32, (16,), 0)

    def _fire_start(a_ref, f_ref):
        for g in range(8):
            sl16 = pl.ds(g * 16, 16)
            csrc_f[sl16] = csrc[sl16]
            cdst_f[sl16] = cdst[sl16]
            ceid_f[sl16] = ceid[sl16]
        pltpu.async_copy(f_ref.at[ceid_f], msg_buf, sem_f)
        pltpu.async_copy(a_ref.at[csrc_f], gat_buf, sem_a)
        for g in range(8):
            lo = pl.ds(g * 16, 16)
            hi = pl.ds(KF + g * 16, 16)
            csrc[lo] = csrc[hi]
            cdst[lo] = cdst[hi]
            ceid[lo] = ceid[hi]

    def _fire_finish(a_ref, f_ref):
        pltpu.make_async_copy(f_ref.at[ceid_f], msg_buf, sem_f).wait()
        pltpu.make_async_copy(a_ref.at[csrc_f], gat_buf, sem_a).wait()

        def _row(r, rc):
            for rr in range(4):
                for g2 in range(4):
                    cols = pl.ds(g2 * 16, 16)
                    x = msg_buf[4 * r + rr, cols] + gat_buf[4 * r + rr, cols]
                    e = jnp.exp(x + x)
                    msg_buf[4 * r + rr, cols] = 1.0 - 2.0 / (e + 1.0)
            return rc
        lax.fori_loop(0, KF // 4, _row, 0)
        pltpu.sync_copy(msg_buf, acc.at[cdst_f], add=True)

    def _process(a_ref, f_ref, src_ref, dst_ref, n_edges, op_dst):
        if op_dst:
            per = n_edges // 16
            start = s * per
        else:
            per = n_edges // 32
            start = (c * 16 + s) * per
        nch = per // KS

        def _scan(b, off, carry):
            fill, pending = carry
            for g in range(8):
                sl16 = pl.ds(g * 16, 16)
                d = dbuf[b, sl16]
                srcv = sbuf[b, sl16]
                if op_dst:
                    dloc = d - base
                    m = (dloc >= 0) & (dloc < HALF)
                else:
                    dloc = d + DEVB
                    m = (d >= 0) & (d < N_DEV)
                eidv = off + g * 16 + lane
                plsc.store_compressed(csrc.at[pl.ds(fill, 16)], srcv, mask=m)
                plsc.store_compressed(cdst.at[pl.ds(fill, 16)], dloc, mask=m)
                plsc.store_compressed(ceid.at[pl.ds(fill, 16)], eidv, mask=m)
                fill = fill + jnp.sum(m.astype(jnp.int32))
            do_fire = fill >= KF

            @pl.when(do_fire)
            def _():
                pl.when(pending == 1)(lambda: _fire_finish(a_ref, f_ref))
                _fire_start(a_ref, f_ref)
            return (jnp.where(do_fire, fill - KF, fill),
                    jnp.where(do_fire, 1, pending))

        sems = ((sem_s0, sem_d0), (sem_s1, sem_d1),
                (sem_s2, sem_d2), (sem_s3, sem_d3))

        def _start(b, off):
            sb, db = sems[b]
            pltpu.async_copy(src_ref.at[pl.ds(off, KS)], sbuf.at[b], sb)
            pltpu.async_copy(dst_ref.at[pl.ds(off, KS)], dbuf.at[b], db)

        def _wait(b):
            sb, db = sems[b]
            pltpu.make_async_copy(src_ref.at[pl.ds(0, KS)], sbuf.at[b], sb).wait()
            pltpu.make_async_copy(dst_ref.at[pl.ds(0, KS)], dbuf.at[b], db).wait()

        for b0 in range(min(3, nch)):
            _start(b0, start + b0 * KS)
        zero2 = (jnp.int32(0), jnp.int32(0))
        nq, tail = nch // 4, nch % 4

        def _quad(k, carry):
            j0 = 4 * k
            for b in range(4):
                _wait(b)
                nxt = j0 + b + 3
                pl.when(nxt < nch)(
                    lambda nxt=nxt, b=b: _start((b + 3) % 4, start + nxt * KS))
                carry = _scan(b, start + (j0 + b) * KS, carry)
            return carry
        carry = zero2
        if nq:
            carry = lax.fori_loop(0, nq, _quad, zero2)
        for t in range(tail):
            j = 4 * nq + t
            _wait(j % 4)
            if j + 3 < nch:
                _start((j + 3) % 4, start + (j + 3) * KS)
            carry = _scan(j % 4, start + j * KS, carry)
        fill, pending = carry

        pl.when(pending == 1)(lambda: _fire_finish(a_ref, f_ref))
        fv = lax.broadcast(fill, (16,))
        dummy_d = DUMMY_OP if op_dst else DUMMY_DEV
        for g in range(8):
            sl16 = pl.ds(g * 16, 16)
            m2 = (g * 16 + lane) < fv
            csrc[sl16] = jnp.where(m2, csrc[sl16], 0)
            cdst[sl16] = jnp.where(m2, cdst[sl16], dummy_d)
            ceid[sl16] = jnp.where(m2, ceid[sl16], 0)
        _fire_start(a_ref, f_ref)
        _fire_finish(a_ref, f_ref)

    _process(a_prev, f_prev, sp, dp, ETP, True)
    _process(a_succ, f_succ, su, du, ETP, True)
    _process(a_serve, f_serve, sv, dv, EPP, True)
    _process(a_place, f_place, spl, dpl, EPP, False)
    _process(a_link, f_link, sl, dl, ELP, False)

    plsc.subcore_barrier()

    r0 = s * (HALF // 16)

    def _wb(kk, carry):
        pltpu.sync_copy(acc.at[pl.ds(r0 + kk * 98, 98)], out_buf)
        pltpu.sync_copy(out_buf, op_out.at[pl.ds(base + r0 + kk * 98, 98)])
        return carry
    lax.fori_loop(0, 16, _wb, 0)
    pltpu.sync_copy(acc.at[pl.ds(DEVB + s * 16, 16)], out_buf.at[pl.ds(0, 16)])
    pltpu.sync_copy(out_buf.at[pl.ds(0, 16)], dev_out.at[c, pl.ds(s * 16, 16)])


def _make_sc_layer():
    mesh = plsc.VectorSubcoreMesh(
        core_axis_name="c", subcore_axis_name="s", num_cores=2, num_subcores=16)
    return pl.kernel(
        _sc_body,
        out_type=[
            jax.ShapeDtypeStruct((NOP_P, H), F32),
            jax.ShapeDtypeStruct((2, N_DEV, H), F32),
        ],
        mesh=mesh,
        compiler_params=pltpu.CompilerParams(
            use_tc_tiling_on_sc=False, needs_layout_passes=False),
        scratch_types=[
            pltpu.VMEM_SHARED((ACC_ROWS, H), F32),
            pltpu.VMEM((4, KS), jnp.int32),
            pltpu.VMEM((4, KS), jnp.int32),
            pltpu.VMEM((2 * KF,), jnp.int32),
            pltpu.VMEM((2 * KF,), jnp.int32),
            pltpu.VMEM((2 * KF,), jnp.int32),
            pltpu.VMEM((KF,), jnp.int32),
            pltpu.VMEM((KF,), jnp.int32),
            pltpu.VMEM((KF,), jnp.int32),
            pltpu.VMEM((KF, H), F32),
            pltpu.VMEM((KF, H), F32),
            pltpu.VMEM((98, H), F32),
            pltpu.SemaphoreType.DMA,
            pltpu.SemaphoreType.DMA,
            pltpu.SemaphoreType.DMA,
            pltpu.SemaphoreType.DMA,
            pltpu.SemaphoreType.DMA,
            pltpu.SemaphoreType.DMA,
            pltpu.SemaphoreType.DMA,
            pltpu.SemaphoreType.DMA,
            pltpu.SemaphoreType.DMA,
            pltpu.SemaphoreType.DMA,
        ],
    )


def _pad_rows(x, n):
    return jnp.pad(x, ((0, n - x.shape[0]),) + ((0, 0),) * (x.ndim - 1))


def _pad_edges(e, n):
    src = jnp.pad(e[0], (0, n - e.shape[1]))
    dst = jnp.pad(e[1], (0, n - e.shape[1]), constant_values=PAD_DST)
    return src.astype(jnp.int32), dst.astype(jnp.int32)


def kernel(op_feats, device_feats, tensor_feats, link_feats, placement_feats,
           params, op_types, prev_edges, succ_edges, link_edges, place_edges,
           serve_edges):
    p = params
    gconv = p["gconv"]

    opf = _pad_rows(op_feats, NOP_P)
    types3 = jnp.pad(op_types.astype(jnp.int32), (0, NOP_P - N_OP)).reshape(GO, 1, BO)
    tf = _pad_rows(tensor_feats, ETP)
    pf = _pad_rows(placement_feats, EPP)
    sp, dp = _pad_edges(prev_edges, ETP)
    su, du = _pad_edges(succ_edges, ETP)
    spl, dpl = _pad_edges(place_edges, EPP)
    sv, dv = _pad_edges(serve_edges, EPP)
    sl, dl = _pad_edges(link_edges, ELP)

    r1 = lambda b: b.reshape(1, -1)

    op_h = pl.pallas_call(
        _oph_body,
        grid=(GO,),
        in_specs=[_rows((BO, D_OP)), _rows((1, 1, BO)), _full((VOCAB, EMB)),
                  _full((D_OP, H)), _full((EMB, H)), _full((1, H))],
        out_specs=_rows((BO, H)),
        out_shape=jax.ShapeDtypeStruct((NOP_P, H), F32),
    )(opf, types3, p["emb"], p["op_trans"]["W"][:D_OP],
      p["op_trans"]["W"][D_OP:], r1(p["op_trans"]["b"]))

    dev_h = pl.pallas_call(
        _devh_body,
        in_specs=[_full((N_DEV, D_DEV)), _full((D_DEV, H)), _full((1, H))],
        out_specs=_full((N_DEV, H)),
        out_shape=jax.ShapeDtypeStruct((N_DEV, H), F32),
    )(device_feats, p["dev_trans"]["W"], r1(p["dev_trans"]["b"]))

    def stackW(et):
        return (jnp.stack([gconv[l][et]["W"][H:] for l in range(3)]),
                jnp.stack([gconv[l][et]["b"].reshape(1, H) for l in range(3)]))

    wf_prev, bf_prev = stackW("prev")
    wf_succ, bf_succ = stackW("succ")
    wf_place, bf_place = stackW("place")
    wf_serve, bf_serve = stackW("serve")
    wf_link, bf_link = stackW("link")

    et = p["edge_trans"]
    BH = BF // 2

    def fspecs(n_rows):
        return dict(
            grid=(n_rows // BF,),
            in_specs=[_rows((BH, 2 * D_E)),
                      _full((2 * D_E, 2 * EH)), _full((1, 2 * EH)),
                      _full((2 * D_E, 2 * EH)), _full((1, 2 * EH)),
                      _full((3, 2 * EH, 2 * H)), _full((3, 1, 2 * H)),
                      _full((3, 2 * EH, 2 * H)), _full((3, 1, 2 * H))],
            out_specs=[_rows((BH, 2 * H))] * 6,
            out_shape=[jax.ShapeDtypeStruct((n_rows // 2, 2 * H), F32)] * 6,
        )

    def bdstack(w3):
        return jnp.stack([_bd(w3[l]) for l in range(3)])

    def b2stack(b3):
        return jnp.stack([_b2(b3[l]) for l in range(3)])

    f_t = pl.pallas_call(_f2_body, **fspecs(ETP))(
        tf.reshape(ETP // 2, 2 * D_E),
        _bd(et["prev"]["W"]), _b2(r1(et["prev"]["b"])),
        _bd(et["succ"]["W"]), _b2(r1(et["succ"]["b"])),
        bdstack(wf_prev), b2stack(bf_prev), bdstack(wf_succ), b2stack(bf_succ))
    f_prev = [o.reshape(ETP, H) for o in f_t[:3]]
    f_succ = [o.reshape(ETP, H) for o in f_t[3:]]

    f_p = pl.pallas_call(_f2_body, **fspecs(EPP))(
        pf.reshape(EPP // 2, 2 * D_E),
        _bd(et["place"]["W"]), _b2(r1(et["place"]["b"])),
        _bd(et["serve"]["W"]), _b2(r1(et["serve"]["b"])),
        bdstack(wf_place), b2stack(bf_place), bdstack(wf_serve), b2stack(bf_serve))
    f_place = [o.reshape(EPP, H) for o in f_p[:3]]
    f_serve = [o.reshape(EPP, H) for o in f_p[3:]]

    f_link_r = pl.pallas_call(
        _f1_body,
        in_specs=[_full((ELP // 2, 2 * D_E)), _full((2 * D_E, 2 * EH)),
                  _full((1, 2 * EH)), _full((3, 2 * EH, 2 * H)), _full((3, 1, 2 * H))],
        out_specs=[_full((ELP // 2, 2 * H))] * 3,
        out_shape=[jax.ShapeDtypeStruct((ELP // 2, 2 * H), F32)] * 3,
    )(link_feats.reshape(ELP // 2, 2 * D_E), _bd(et["link"]["W"]),
      _b2(r1(et["link"]["b"])), bdstack(wf_link), b2stack(bf_link))
    f_link = [o.reshape(ELP, H) for o in f_link_r]

    sc_layer = _make_sc_layer()

    op_h2 = op_h.reshape(NOP_P // 2, 2 * H)
    for l in range(3):
        lp = gconv[l]
        a2 = pl.pallas_call(
            _aop_body,
            grid=(GO,),
            in_specs=[_rows((BO // 2, 2 * H))] + [_full((2 * H, 2 * H))] * 3,
            out_specs=[_rows((BO // 2, 2 * H))] * 3,
            out_shape=[jax.ShapeDtypeStruct((NOP_P // 2, 2 * H), F32)] * 3,
        )(op_h2, _bd(lp["prev"]["W"][:H]), _bd(lp["succ"]["W"][:H]),
          _bd(lp["place"]["W"][:H]))
        a_prev, a_succ, a_place = (o.reshape(NOP_P, H) for o in a2)

        a_serve, a_link = pl.pallas_call(
            _adev_body,
            in_specs=[_full((N_DEV, H))] + [_full((H, H))] * 2,
            out_specs=[_full((N_DEV, H))] * 2,
            out_shape=[jax.ShapeDtypeStruct((N_DEV, H), F32)] * 2,
        )(dev_h, lp["serve"]["W"][:H], lp["link"]["W"][:H])

        op_sum, dev_sum = sc_layer(
            a_prev, a_succ, a_place, a_serve, a_link,
            f_prev[l], f_succ[l], f_place[l], f_serve[l], f_link[l],
            sp, dp, su, du, spl, dpl, sv, dv, sl, dl)

        op_h2 = pl.pallas_call(
            _updop_body,
            grid=(GO,),
            in_specs=[_rows((BO // 2, 2 * H)), _rows((BO // 2, 2 * H)),
                      _full((2 * H, 2 * H)), _full((1, 2 * H))],
            out_specs=_rows((BO // 2, 2 * H)),
            out_shape=jax.ShapeDtypeStruct((NOP_P // 2, 2 * H), F32),
        )(op_h2, op_sum.reshape(NOP_P // 2, 2 * H), _bd(lp["op_final"]["W"]),
          _b2(r1(lp["op_final"]["b"])))

        dev_h = pl.pallas_call(
            _upddev_body,
            in_specs=[_full((N_DEV, H)), _full((2, N_DEV, H)), _full((H, H)), _full((1, H))],
            out_specs=_full((N_DEV, H)),
            out_shape=jax.ShapeDtypeStruct((N_DEV, H), F32),
        )(dev_h, dev_sum, lp["device_final"]["W"], r1(lp["device_final"]["b"]))

    out = pl.pallas_call(
        _final_body,
        grid=(GO,),
        in_specs=[_rows((BO, H)), _full((N_DEV, H))],
        out_specs=_rows((BO, N_DEV)),
        out_shape=jax.ShapeDtypeStruct((NOP_P, N_DEV), F32),
    )(op_h2.reshape(NOP_P, H), dev_h)
    return out[:N_OP]
